# baseline (speedup 1.0000x reference)
import functools

import jax
import jax.numpy as jnp
import numpy as np
from jax import lax
from jax.experimental import pallas as pl
from jax.experimental.pallas import tpu as pltpu
from jax.experimental.pallas import tpu_sc as plsc

_VOCAB = 50257
_COMPRESSED = 38697
_M = 100003
_NUM_HEADS = 8
_ORDERS = (2, 3)
_DIM = 16

_B, _S = 4, 4096
_NW = 32
_POS_PER_W = (_B * _S) // _NW
_ROWS_PER_W = _POS_PER_W * 16
_ROW_PAD = _S + 8
_TOK_COPY = _POS_PER_W + 8


def _combo_params():
    out = []
    for oi, n in enumerate(_ORDERS):
        rng = np.random.RandomState(42 + n)
        coeffs = rng.randint(1, _M, size=(_NUM_HEADS, n))
        seeds = rng.randint(0, _M, size=(_NUM_HEADS,))
        for h in range(_NUM_HEADS):
            c3 = [0] * (3 - n) + [int(v) for v in coeffs[h]]
            out.append((c3, int(seeds[h]), (oi * _NUM_HEADS + h) * _M))
    return out


_COMBOS = _combo_params()


_I16 = np.int32(16)
_M32 = np.int32(_M)


def _hash_vec(xs, c3, seed):
    lo = None
    hi = None
    for c, x in zip(c3, xs):
        if c == 0:
            continue
        a, b = c >> 16, c & 0xFFFF
        t = x * np.int32(b)
        r = t & np.int32(0xFFFF)
        q = lax.shift_right_logical(t, _I16)
        term = q + x if a else q
        lo = r if lo is None else lo + r
        hi = term if hi is None else hi + term
    hi = hi + lax.shift_right_logical(lo, _I16)
    lo = lo & np.int32(0xFFFF)
    lo = lo ^ np.int32(seed & 0xFFFF)
    if seed >> 16:
        hi = hi ^ np.int32(1)
    h = lax.rem(hi, _M32)
    h = lax.rem(h * np.int32(256), _M32)
    h = lax.rem(h * np.int32(256), _M32)
    return lax.rem(h + lo, _M32)


def _sc_body(tok_hbm, table_hbm, out_hbm, tok_v, idx_v, rows_v, sem):
    wid = lax.axis_index("s") * np.int32(2) + lax.axis_index("c")
    b = wid // np.int32(8)
    s0 = (wid % np.int32(8)) * np.int32(_POS_PER_W)

    pltpu.sync_copy(tok_hbm.at[pl.ds(b * np.int32(_ROW_PAD) + s0, _TOK_COPY)],
                    tok_v.at[pl.ds(0, _TOK_COPY)])

    def _compress(_, off):
        sl = pl.ds(off, 16)
        tok_v[sl] = lax.rem(tok_v[sl], np.int32(_COMPRESSED))
        return off + np.int32(16)

    lax.fori_loop(0, (_TOK_COPY + 8) // 16, _compress, np.int32(0))

    iota = lax.iota(jnp.int32, 16)

    def _hash_chunk(_, p0):
        xm2 = plsc.load_gather(tok_v, [iota + p0])
        xm1 = plsc.load_gather(tok_v, [iota + (p0 + np.int32(1))])
        x0 = plsc.load_gather(tok_v, [iota + (p0 + np.int32(2))])
        xs = (xm2, xm1, x0)
        fbase = (p0 + iota) * np.int32(16)
        for t, (c3, seed, off) in enumerate(_COMBOS):
            f = fbase + np.int32(t)
            plsc.store_scatter(
                idx_v,
                [lax.shift_right_logical(f, np.int32(7)), f & np.int32(127)],
                _hash_vec(xs, c3, seed) + np.int32(off))
        return p0 + np.int32(16)

    lax.fori_loop(0, _POS_PER_W // 16, _hash_chunk, np.int32(0))

    out_base = wid * np.int32(_ROWS_PER_W)
    for g in range(4):
        copies = [
            pltpu.async_copy(table_hbm.at[idx_v.at[np.int32(g * 16 + j)]],
                             rows_v.at[pl.ds(j * 128, 128)], sem)
            for j in range(16)
        ]
        for cp in copies:
            cp.wait()
        pltpu.sync_copy(rows_v, out_hbm.at[pl.ds(out_base + np.int32(g * 2048), 2048)])


_sc_lookup = functools.partial(
    pl.kernel,
    out_type=jax.ShapeDtypeStruct((_B * _S * 16, _DIM), jnp.float32),
    mesh=plsc.VectorSubcoreMesh(core_axis_name="c", subcore_axis_name="s"),
    scratch_types=[
        pltpu.VMEM((_TOK_COPY + 8,), jnp.int32),
        pltpu.VMEM((64, 128), jnp.int32),
        pltpu.VMEM((2048, _DIM), jnp.float32),
        pltpu.SemaphoreType.DMA,
    ],
    compiler_params=pltpu.CompilerParams(needs_layout_passes=False, use_tc_tiling_on_sc=False),
)(_sc_body)


def kernel(token_ids, tables):
    tok32 = token_ids.astype(jnp.int32)
    tok_pad = jnp.pad(tok32, ((0, 0), (2, _ROW_PAD - _S - 2))).reshape(-1)
    table_flat = tables.reshape(len(_ORDERS) * _NUM_HEADS * _M, _DIM)
    out = _sc_lookup(tok_pad, table_flat)
    return out.reshape(_B, _S, 16 * _DIM)


# speedup vs baseline: 2.0487x; 2.0487x over previous
import functools

import jax
import jax.numpy as jnp
import numpy as np
from jax import lax
from jax.experimental import pallas as pl
from jax.experimental.pallas import tpu as pltpu
from jax.experimental.pallas import tpu_sc as plsc

_VOCAB = 50257
_COMPRESSED = 38697
_M = 100003
_NUM_HEADS = 8
_ORDERS = (2, 3)
_DIM = 16

_B, _S = 4, 4096
_NW = 32
_POS_PER_W = (_B * _S) // _NW
_ROWS_PER_W = _POS_PER_W * 16
_ROW_PAD = _S + 8
_TOK_COPY = _POS_PER_W + 8
_TROWS = len(_ORDERS) * _NUM_HEADS * _M // 8
_C1 = (1 << 25) % _M
_INV_M = np.float32(1.0) / np.float32(_M)


def _combo_params():
    out = []
    for oi, n in enumerate(_ORDERS):
        rng = np.random.RandomState(42 + n)
        coeffs = rng.randint(1, _M, size=(_NUM_HEADS, n))
        seeds = rng.randint(0, _M, size=(_NUM_HEADS,))
        for h in range(_NUM_HEADS):
            c3 = [0] * (3 - n) + [int(v) for v in coeffs[h]]
            out.append((c3, int(seeds[h]), (oi * _NUM_HEADS + h) * _M))
    return out


_COMBOS = _combo_params()
_I16 = np.int32(16)


def _hash_vec(xs, c3, seed):
    lo = None
    hi = None
    for c, x in zip(c3, xs):
        if c == 0:
            continue
        a, b = c >> 16, c & 0xFFFF
        t = x * np.int32(b)
        r = t & np.int32(0xFFFF)
        q = lax.shift_right_logical(t, _I16)
        term = q + x if a else q
        lo = r if lo is None else lo + r
        hi = term if hi is None else hi + term
    hi = hi + lax.shift_right_logical(lo, _I16)
    lo = lo & np.int32(0xFFFF)
    lo = lo ^ np.int32(seed & 0xFFFF)
    if seed >> 16:
        hi = hi ^ np.int32(1)
    u = lax.shift_right_logical(hi, np.int32(9))
    v = hi & np.int32(511)
    s = u * np.int32(_C1) + lax.shift_left(v, _I16) + lo
    qe = (s.astype(jnp.float32) * _INV_M).astype(jnp.int32)
    r = s - qe * np.int32(_M)
    r = jnp.where(r < 0, r + np.int32(_M), r)
    return jnp.where(r >= np.int32(_M), r - np.int32(_M), r)


def _sc_body(tok_hbm, table_hbm, out_hbm, tok_v, idx_v, sub_v, big_v,
             pack_v, sem):
    wid = lax.axis_index("s") * np.int32(2) + lax.axis_index("c")
    b = wid // np.int32(8)
    s0 = (wid % np.int32(8)) * np.int32(_POS_PER_W)

    pltpu.sync_copy(tok_hbm.at[pl.ds(b * np.int32(_ROW_PAD) + s0, _TOK_COPY)],
                    tok_v.at[pl.ds(0, _TOK_COPY)])

    def _compress(_, off):
        sl = pl.ds(off, 16)
        x = tok_v[sl]
        tok_v[sl] = jnp.where(x >= np.int32(_COMPRESSED),
                              x - np.int32(_COMPRESSED), x)
        return off + np.int32(16)

    lax.fori_loop(0, (_TOK_COPY + 8) // 16, _compress, np.int32(0))

    iota = lax.iota(jnp.int32, 16)

    def _hash_chunk(_, p0):
        xm2 = plsc.load_gather(tok_v, [iota + p0])
        xm1 = plsc.load_gather(tok_v, [iota + (p0 + np.int32(1))])
        x0 = plsc.load_gather(tok_v, [iota + (p0 + np.int32(2))])
        xs = (xm2, xm1, x0)
        fbase = (p0 + iota) * _I16
        for t, (c3, seed, off) in enumerate(_COMBOS):
            slot = fbase + np.int32(t)
            srow = lax.shift_right_logical(slot, np.int32(7))
            scol = slot & np.int32(127)
            f = _hash_vec(xs, c3, seed) + np.int32(off)
            plsc.store_scatter(idx_v, [srow, scol],
                               lax.shift_right_logical(f, np.int32(3)))
            plsc.store_scatter(sub_v, [srow, scol],
                               lax.shift_left(f, np.int32(4)) & np.int32(127))
        return p0 + _I16

    lax.fori_loop(0, _POS_PER_W // 16, _hash_chunk, np.int32(0))

    out_base = wid * np.int32(_ROWS_PER_W * _DIM)

    def _group(_, j):
        cp = pltpu.async_copy(table_hbm.at[idx_v.at[j]], big_v, sem)
        cp.wait()
        for i0 in range(8):
            s16 = sub_v[j, pl.ds(i0 * 16, 16)]
            rows = iota + np.int32(i0 * 16)
            base = rows * _I16
            for d in range(16):
                vals = plsc.load_gather(big_v, [rows, s16 + np.int32(d)])
                plsc.store_scatter(pack_v, [base + np.int32(d)], vals)
        pltpu.sync_copy(
            pack_v, out_hbm.at[pl.ds(out_base + j * np.int32(2048), 2048)])
        return j + np.int32(1)

    lax.fori_loop(0, 64, _group, np.int32(0))


_sc_lookup = functools.partial(
    pl.kernel,
    out_type=jax.ShapeDtypeStruct((_B * _S * 16 * _DIM,), jnp.float32),
    mesh=plsc.VectorSubcoreMesh(core_axis_name="c", subcore_axis_name="s"),
    scratch_types=[
        pltpu.VMEM((_TOK_COPY + 8,), jnp.int32),
        pltpu.VMEM((64, 128), jnp.int32),
        pltpu.VMEM((64, 128), jnp.int32),
        pltpu.VMEM((128, 128), jnp.float32),
        pltpu.VMEM((2048,), jnp.float32),
        pltpu.SemaphoreType.DMA,
    ],
    compiler_params=pltpu.CompilerParams(needs_layout_passes=False,
                                         use_tc_tiling_on_sc=False),
)(_sc_body)


def kernel(token_ids, tables):
    tok32 = token_ids.astype(jnp.int32)
    tok_pad = jnp.pad(tok32, ((0, 0), (2, _ROW_PAD - _S - 2))).reshape(-1)
    table128 = tables.reshape(_TROWS, 8 * _DIM)
    out = _sc_lookup(tok_pad, table128)
    return out.reshape(_B, _S, 16 * _DIM)
